# Initial kernel scaffold; baseline (speedup 1.0000x reference)
import jax
import jax.numpy as jnp
from jax.experimental import pallas as pl


def kernel(X, transformation_scale, transformation_translate):
    raise NotImplementedError("write your pallas kernel here")


# speedup vs baseline: 1.5765x; 1.5765x over previous
import jax
import jax.numpy as jnp
from jax import lax
from jax.experimental import pallas as pl
from jax.experimental.pallas import tpu as pltpu
from jax.experimental.pallas import tpu_sc as plsc

_OUT_H, _OUT_W = 112, 112
_B, _H, _W, _C = 8, 224, 224, 96
_NC, _NS = 2, 16
_NW = _NC * _NS
_RPW = (_B * _OUT_H) // _NW
_WPB = _OUT_H // _RPW
_ROWLEN = _W * _C
_OROW = _OUT_W * _C
_L = 16
_NG = _OUT_W // _L


def _rne_bf16(v):
    bits = plsc.bitcast(v, jnp.int32)
    r = bits + jnp.int32(0x7FFF) + ((bits >> 16) & 1)
    r = jnp.bitwise_and(r, jnp.int32(-65536))
    return plsc.bitcast(r, jnp.float32)


def _body(x_hbm, sc_hbm, tr_hbm, out_hbm,
          ya0, yb0, ya1, yb1, obuf0, obuf1, idx0, idx1,
          x0i, x1i, wx0v, wx1v, sbuf, tbuf,
          insem0, insem1, outsem0, outsem1):
    cid = lax.axis_index("c")
    sid = lax.axis_index("s")
    wid = sid * _NC + cid
    b = wid // _WPB
    i0 = (wid % _WPB) * _RPW

    pltpu.sync_copy(sc_hbm, sbuf)
    pltpu.sync_copy(tr_hbm, tbuf)
    lanes = lax.iota(jnp.int32, _L)
    z16 = jnp.zeros((_L,), jnp.int32)
    bv = jnp.full((_L,), b, jnp.int32)
    s_spl = _rne_bf16(plsc.load_gather(sbuf, [bv]))
    tx_spl = _rne_bf16(plsc.load_gather(tbuf, [bv * 2]))
    ty_spl = _rne_bf16(plsc.load_gather(tbuf, [bv * 2 + 1]))

    half = jnp.float32(0.5)
    one = jnp.float32(1.0)
    step = jnp.float32(2.0 / 111.0)

    for g in range(_NG):
        jf = (lanes + g * _L).astype(jnp.float32)
        xg = _rne_bf16(jf * step - one)
        x = half * (s_spl * xg + tx_spl + one) * jnp.float32(_W)
        x0t = x.astype(jnp.int32)
        x0c = jnp.clip(x0t, 0, _W - 1)
        x1c = jnp.clip(x0t + 1, 0, _W - 1)
        x0i[pl.ds(g * _L, _L)] = x0c
        x1i[pl.ds(g * _L, _L)] = x1c
        wx0v[pl.ds(g * _L, _L)] = x1c.astype(jnp.float32) - x
        wx1v[pl.ds(g * _L, _L)] = x - x0c.astype(jnp.float32)

    def yrows(k):
        iv = jnp.full((_L,), i0 + k, jnp.int32).astype(jnp.float32)
        yg = _rne_bf16(iv * step - one)
        y = half * (s_spl * yg + ty_spl + one) * jnp.float32(_H)
        y0t = y.astype(jnp.int32)
        y0c = jnp.clip(y0t, 0, _H - 1)
        y1c = jnp.clip(y0t + 1, 0, _H - 1)
        return y, y0c, y1c

    def issue_in(k, idxb, ya, yb, insem):
        _, y0c, y1c = yrows(k)
        idxb[...] = jnp.where(lanes < 8, y0c, y1c) + bv * _H
        pltpu.async_copy(x_hbm.at[idxb.at[pl.ds(0, 1)]], ya, insem)
        pltpu.async_copy(x_hbm.at[idxb.at[pl.ds(8, 1)]], yb, insem)

    def wait_in(idxb, ya, yb, insem):
        pltpu.make_async_copy(x_hbm.at[idxb.at[pl.ds(0, 1)]], ya, insem).wait()
        pltpu.make_async_copy(x_hbm.at[idxb.at[pl.ds(8, 1)]], yb, insem).wait()

    def wait_out(obuf, outsem):
        pltpu.make_async_copy(obuf, out_hbm.at[0], outsem).wait()

    def compute_row(k, ya, yb, obuf):
        y, y0c, y1c = yrows(k)
        wy0 = y1c.astype(jnp.float32) - y
        wy1 = y - y0c.astype(jnp.float32)

        @pl.loop(0, _NG)
        def _gloop(g):
            g16 = g * _L
            x0c = x0i[pl.ds(g16, _L)]
            x1c = x1i[pl.ds(g16, _L)]
            wx0 = wx0v[pl.ds(g16, _L)]
            wx1 = wx1v[pl.ds(g16, _L)]
            wa = wx0 * wy0
            wb = wx0 * wy1
            wc = wx1 * wy0
            wd = wx1 * wy1
            u0 = x0c * _C
            u1 = x1c * _C
            jb = (lanes + g16) * _C

            @pl.loop(0, _C, unroll=8)
            def _cloop(c):
                cs = jnp.full((_L,), c, jnp.int32)
                u0c = u0 + cs
                u1c = u1 + cs
                va = plsc.load_gather(ya, [z16, u0c])
                vb = plsc.load_gather(yb, [z16, u0c])
                vc = plsc.load_gather(ya, [z16, u1c])
                vd = plsc.load_gather(yb, [z16, u1c])
                r = wa * va + wb * vb + wc * vc + wd * vd
                plsc.store_scatter(obuf, [jb + cs], r)

    bufs = ((ya0, yb0, obuf0, idx0, insem0, outsem0),
            (ya1, yb1, obuf1, idx1, insem1, outsem1))

    @pl.loop(0, _RPW, step=2)
    def _rowloop(k0):
        for p in range(2):
            ya, yb, obuf, idxb, insem, outsem = bufs[p]
            k = k0 + p
            issue_in(k, idxb, ya, yb, insem)
            wait_in(idxb, ya, yb, insem)
            compute_row(k, ya, yb, obuf)
            pltpu.async_copy(obuf, out_hbm.at[wid * _RPW + k], outsem)
            wait_out(obuf, outsem)


@jax.jit
def kernel(X, transformation_scale, transformation_translate):
    x2 = X.reshape(_B * _H, _ROWLEN)
    scale16 = jnp.concatenate(
        [transformation_scale[:, 0], jnp.zeros((8,), jnp.float32)])
    transl16 = transformation_translate.reshape(16)

    mesh = plsc.VectorSubcoreMesh(core_axis_name="c", subcore_axis_name="s")
    run = pl.kernel(
        _body,
        out_type=jax.ShapeDtypeStruct((_B * _OUT_H, _OROW), jnp.float32),
        mesh=mesh,
        compiler_params=pltpu.CompilerParams(needs_layout_passes=False),
        scratch_types=[
            pltpu.VMEM((1, _ROWLEN), jnp.float32),
            pltpu.VMEM((1, _ROWLEN), jnp.float32),
            pltpu.VMEM((1, _ROWLEN), jnp.float32),
            pltpu.VMEM((1, _ROWLEN), jnp.float32),
            pltpu.VMEM((_OROW,), jnp.float32),
            pltpu.VMEM((_OROW,), jnp.float32),
            pltpu.VMEM((_L,), jnp.int32),
            pltpu.VMEM((_L,), jnp.int32),
            pltpu.VMEM((_OUT_W,), jnp.int32),
            pltpu.VMEM((_OUT_W,), jnp.int32),
            pltpu.VMEM((_OUT_W,), jnp.float32),
            pltpu.VMEM((_OUT_W,), jnp.float32),
            pltpu.VMEM((_L,), jnp.float32),
            pltpu.VMEM((_L,), jnp.float32),
            pltpu.SemaphoreType.DMA,
            pltpu.SemaphoreType.DMA,
            pltpu.SemaphoreType.DMA,
            pltpu.SemaphoreType.DMA,
        ],
    )
    out = run(x2, scale16, transl16)
    return out.reshape(_B, _OUT_H, _OUT_W, _C)


# speedup vs baseline: 1.6401x; 1.0403x over previous
import jax
import jax.numpy as jnp
from jax import lax
from jax.experimental import pallas as pl
from jax.experimental.pallas import tpu as pltpu
from jax.experimental.pallas import tpu_sc as plsc

_OUT_H, _OUT_W = 112, 112
_B, _H, _W, _C = 8, 224, 224, 96
_NC, _NS = 2, 16
_NW = _NC * _NS
_RPW = (_B * _OUT_H) // _NW
_WPB = _OUT_H // _RPW
_ROWLEN = _W * _C
_OROW = _OUT_W * _C
_L = 16
_NG = _OUT_W // _L


def _rne_bf16(v):
    bits = plsc.bitcast(v, jnp.int32)
    r = bits + jnp.int32(0x7FFF) + ((bits >> 16) & 1)
    r = jnp.bitwise_and(r, jnp.int32(-65536))
    return plsc.bitcast(r, jnp.float32)


def _body(x_hbm, sc_hbm, tr_hbm, out_hbm,
          ya0, yb0, ya1, yb1, obuf0, obuf1, idx0, idx1,
          x0i, x1i, wx0v, wx1v, sbuf, tbuf,
          insem0, insem1, outsem0, outsem1):
    cid = lax.axis_index("c")
    sid = lax.axis_index("s")
    wid = sid * _NC + cid
    b = wid // _WPB
    i0 = (wid % _WPB) * _RPW

    pltpu.sync_copy(sc_hbm, sbuf)
    pltpu.sync_copy(tr_hbm, tbuf)
    lanes = lax.iota(jnp.int32, _L)
    z16 = jnp.zeros((_L,), jnp.int32)
    bv = jnp.full((_L,), b, jnp.int32)
    s_spl = _rne_bf16(plsc.load_gather(sbuf, [bv]))
    tx_spl = _rne_bf16(plsc.load_gather(tbuf, [bv * 2]))
    ty_spl = _rne_bf16(plsc.load_gather(tbuf, [bv * 2 + 1]))

    half = jnp.float32(0.5)
    one = jnp.float32(1.0)
    step = jnp.float32(2.0 / 111.0)

    for g in range(_NG):
        jf = (lanes + g * _L).astype(jnp.float32)
        xg = _rne_bf16(jf * step - one)
        x = half * (s_spl * xg + tx_spl + one) * jnp.float32(_W)
        x0t = x.astype(jnp.int32)
        x0c = jnp.clip(x0t, 0, _W - 1)
        x1c = jnp.clip(x0t + 1, 0, _W - 1)
        x0i[pl.ds(g * _L, _L)] = x0c
        x1i[pl.ds(g * _L, _L)] = x1c
        wx0v[pl.ds(g * _L, _L)] = x1c.astype(jnp.float32) - x
        wx1v[pl.ds(g * _L, _L)] = x - x0c.astype(jnp.float32)

    def yrows(k):
        iv = jnp.full((_L,), i0 + k, jnp.int32).astype(jnp.float32)
        yg = _rne_bf16(iv * step - one)
        y = half * (s_spl * yg + ty_spl + one) * jnp.float32(_H)
        y0t = y.astype(jnp.int32)
        y0c = jnp.clip(y0t, 0, _H - 1)
        y1c = jnp.clip(y0t + 1, 0, _H - 1)
        return y, y0c, y1c

    def issue_in(k, idxb, ya, yb, insem):
        _, y0c, y1c = yrows(k)
        idxb[...] = jnp.where(lanes < 8, y0c, y1c) + bv * _H
        pltpu.async_copy(x_hbm.at[idxb.at[pl.ds(0, 1)]], ya, insem)
        pltpu.async_copy(x_hbm.at[idxb.at[pl.ds(8, 1)]], yb, insem)

    def wait_in(idxb, ya, yb, insem):
        pltpu.make_async_copy(x_hbm.at[idxb.at[pl.ds(0, 1)]], ya, insem).wait()
        pltpu.make_async_copy(x_hbm.at[idxb.at[pl.ds(8, 1)]], yb, insem).wait()

    def wait_out(obuf, outsem):
        pltpu.make_async_copy(obuf, out_hbm.at[0], outsem).wait()

    def compute_row(k, ya, yb, obuf):
        y, y0c, y1c = yrows(k)
        wy0 = y1c.astype(jnp.float32) - y
        wy1 = y - y0c.astype(jnp.float32)

        @pl.loop(0, _NG)
        def _gloop(g):
            g16 = g * _L
            x0c = x0i[pl.ds(g16, _L)]
            x1c = x1i[pl.ds(g16, _L)]
            wx0 = wx0v[pl.ds(g16, _L)]
            wx1 = wx1v[pl.ds(g16, _L)]
            wa = wx0 * wy0
            wb = wx0 * wy1
            wc = wx1 * wy0
            wd = wx1 * wy1
            u0 = x0c * _C
            u1 = x1c * _C
            jb = (lanes + g16) * _C

            @pl.loop(0, _C, unroll=8)
            def _cloop(c):
                cs = jnp.full((_L,), c, jnp.int32)
                u0c = u0 + cs
                u1c = u1 + cs
                va = plsc.load_gather(ya, [z16, u0c])
                vb = plsc.load_gather(yb, [z16, u0c])
                vc = plsc.load_gather(ya, [z16, u1c])
                vd = plsc.load_gather(yb, [z16, u1c])
                r = wa * va + wb * vb + wc * vc + wd * vd
                plsc.store_scatter(obuf, [jb + cs], r)

    bufs = ((ya0, yb0, obuf0, idx0, insem0, outsem0),
            (ya1, yb1, obuf1, idx1, insem1, outsem1))

    issue_in(0, idx0, ya0, yb0, insem0)
    issue_in(1, idx1, ya1, yb1, insem1)

    @pl.loop(0, _RPW, step=2)
    def _rowloop(k0):
        for p in range(2):
            ya, yb, obuf, idxb, insem, outsem = bufs[p]
            k = k0 + p

            @pl.when(k0 > 0)
            def _():
                wait_out(obuf, outsem)

            wait_in(idxb, ya, yb, insem)
            compute_row(k, ya, yb, obuf)
            pltpu.async_copy(obuf, out_hbm.at[wid * _RPW + k], outsem)

            @pl.when(k + 2 < _RPW)
            def _():
                issue_in(k + 2, idxb, ya, yb, insem)

    wait_out(obuf0, outsem0)
    wait_out(obuf1, outsem1)


@jax.jit
def kernel(X, transformation_scale, transformation_translate):
    x2 = X.reshape(_B * _H, _ROWLEN)
    scale16 = jnp.concatenate(
        [transformation_scale[:, 0], jnp.zeros((8,), jnp.float32)])
    transl16 = transformation_translate.reshape(16)

    mesh = plsc.VectorSubcoreMesh(core_axis_name="c", subcore_axis_name="s")
    run = pl.kernel(
        _body,
        out_type=jax.ShapeDtypeStruct((_B * _OUT_H, _OROW), jnp.float32),
        mesh=mesh,
        compiler_params=pltpu.CompilerParams(needs_layout_passes=False),
        scratch_types=[
            pltpu.VMEM((1, _ROWLEN), jnp.float32),
            pltpu.VMEM((1, _ROWLEN), jnp.float32),
            pltpu.VMEM((1, _ROWLEN), jnp.float32),
            pltpu.VMEM((1, _ROWLEN), jnp.float32),
            pltpu.VMEM((_OROW,), jnp.float32),
            pltpu.VMEM((_OROW,), jnp.float32),
            pltpu.VMEM((_L,), jnp.int32),
            pltpu.VMEM((_L,), jnp.int32),
            pltpu.VMEM((_OUT_W,), jnp.int32),
            pltpu.VMEM((_OUT_W,), jnp.int32),
            pltpu.VMEM((_OUT_W,), jnp.float32),
            pltpu.VMEM((_OUT_W,), jnp.float32),
            pltpu.VMEM((_L,), jnp.float32),
            pltpu.VMEM((_L,), jnp.float32),
            pltpu.SemaphoreType.DMA,
            pltpu.SemaphoreType.DMA,
            pltpu.SemaphoreType.DMA,
            pltpu.SemaphoreType.DMA,
        ],
    )
    out = run(x2, scale16, transl16)
    return out.reshape(_B, _OUT_H, _OUT_W, _C)
